# baseline (speedup 1.0000x reference)
import functools

import jax
import jax.numpy as jnp
from jax.experimental import pallas as pl
from jax.experimental.pallas import tpu as pltpu

_RADIUS = 0.02
_W = 1024
_NW = 2
_Q_SUB = 64


def _csdf_kernel(nc, nsub, qzlo_ref, qzhi_ref, pzlo_ref, pzhi_ref, jl_ref,
                 x_ref, p_ref, out_ref, m_ref, macc_ref):
    b = pl.program_id(0)
    m_ref[0] = jnp.inf
    macc_ref[...] = jnp.full(macc_ref.shape, jnp.inf, jnp.float32)
    q = x_ref[0]

    def block_fold(qs, j, acc):
        pt = p_ref[j]
        d2 = None
        for c3 in range(3):
            diff = qs[:, c3:c3 + 1] - pt[c3:c3 + 1, :]
            sq = diff * diff
            d2 = sq if d2 is None else d2 + sq
        for w in range(_W // 128):
            acc = jnp.minimum(acc, d2[:, w * 128:(w + 1) * 128])
        return acc

    def block_min(qs, j):
        macc_ref[...] = block_fold(qs, j, macc_ref[...])

    mid = nsub // 2
    korder = []
    for d in range(nsub):
        lo, hi = mid - 1 - d // 2, mid + d // 2
        korder.append(hi if d % 2 == 0 else lo)

    for k in korder:
        qzl = qzlo_ref[b, k]
        qzh = qzhi_ref[b, k]
        qs = q[k * _Q_SUB:(k + 1) * _Q_SUB, :]
        jl = jl_ref[b, k]

        acc = macc_ref[...]
        for i in range(_NW):
            acc = block_fold(qs, jl + i, acc)
        macc_ref[...] = acc

        m_ref[0] = jnp.minimum(m_ref[0], jnp.min(acc))

        def body(c, _step=1, _qzl=qzl, _qzh=qzh, _qs=qs):
            j, _ = c
            pzl = pzlo_ref[j]
            pzh = pzhi_ref[j]
            gap = jnp.maximum(jnp.maximum(pzl - _qzh, _qzl - pzh), 0.0)
            pred = gap * gap < m_ref[0]

            @pl.when(pred)
            def _():
                block_min(_qs, j)

            return (j + _step, jnp.logical_not(pred))

        def cond(c):
            j, done = c
            return (j >= 0) & (j < nc) & jnp.logical_not(done)

        jax.lax.while_loop(cond, functools.partial(body, _step=1),
                           (jl + _NW, False))
        jax.lax.while_loop(cond, functools.partial(body, _step=-1),
                           (jl - 1, False))

    out_ref[...] = jnp.full(
        out_ref.shape,
        jnp.sqrt(jnp.maximum(jnp.min(macc_ref[...]), 0.0)) - _RADIUS,
        dtype=out_ref.dtype)


def kernel(x, pcd):
    B, P, _ = x.shape
    N = pcd.shape[0]
    n_pad = -N % _W
    nc = (N + n_pad) // _W
    nsub = P // _Q_SUB

    ps = jax.lax.sort([pcd[:, 2], pcd[:, 0], pcd[:, 1]], num_keys=1)
    pcd_t = jnp.stack([ps[1], ps[2], ps[0]])
    pcd_t = jnp.pad(pcd_t, ((0, 0), (0, n_pad)), mode="edge")
    qs = jax.lax.sort([x[:, :, 2], x[:, :, 0], x[:, :, 1]],
                      dimension=1, num_keys=1)
    x_s = jnp.stack([qs[1], qs[2], qs[0]], axis=-1)

    pz = pcd_t[2]
    pzlo = pz[::_W]
    pzhi = pz[_W - 1::_W]
    qz = x_s[:, :, 2]
    qzlo = qz[:, ::_Q_SUB]
    qzhi = qz[:, _Q_SUB - 1::_Q_SUB]

    qc = 0.5 * (qzlo + qzhi)
    j0 = jnp.minimum(jnp.searchsorted(pzhi, qc.reshape(-1)),
                     nc - 1).astype(jnp.int32).reshape(B, nsub)
    jl = jnp.clip(j0 - _NW // 2, 0, nc - _NW)

    p_chunks = pcd_t.reshape(3, nc, _W).transpose(1, 0, 2)

    grid_spec = pltpu.PrefetchScalarGridSpec(
        num_scalar_prefetch=5,
        grid=(B,),
        in_specs=[
            pl.BlockSpec((1, P, 3), lambda b, *_: (b, 0, 0)),
            pl.BlockSpec((nc, 3, _W), lambda b, *_: (0, 0, 0)),
        ],
        out_specs=pl.BlockSpec((1, 1, 128), lambda b, *_: (b, 0, 0)),
        scratch_shapes=[pltpu.SMEM((1,), jnp.float32),
                        pltpu.VMEM((_Q_SUB, 128), jnp.float32)],
    )
    out = pl.pallas_call(
        functools.partial(_csdf_kernel, nc, nsub),
        grid_spec=grid_spec,
        out_shape=jax.ShapeDtypeStruct((B, 1, 128), jnp.float32),
    )(qzlo, qzhi, pzlo, pzhi, jl, x_s, p_chunks)
    return out[:, 0, 0]


# speedup vs baseline: 1.9734x; 1.9734x over previous
import functools

import jax
import jax.numpy as jnp
from jax.experimental import pallas as pl
from jax.experimental.pallas import tpu as pltpu

_RADIUS = 0.02
_W = 512
_NW = 4
_Q_SUB = 64


def _csdf_kernel(nc, nsub, qzlo_ref, qzhi_ref, pzlo_ref, pzhi_ref, jl_ref,
                 x_ref, p_ref, out_ref, m_ref, macc_ref):
    b = pl.program_id(0)
    m_ref[0] = jnp.inf
    macc_ref[...] = jnp.full(macc_ref.shape, jnp.inf, jnp.float32)
    q = x_ref[0]

    def block_fold(qs, j, acc):
        pt = p_ref[j]
        d2 = None
        for c3 in range(3):
            diff = qs[:, c3:c3 + 1] - pt[c3:c3 + 1, :]
            sq = diff * diff
            d2 = sq if d2 is None else d2 + sq
        for w in range(_W // 128):
            acc = jnp.minimum(acc, d2[:, w * 128:(w + 1) * 128])
        return acc

    def block_min(qs, j):
        macc_ref[...] = block_fold(qs, j, macc_ref[...])

    mid = nsub // 2
    korder = []
    for d in range(nsub):
        lo, hi = mid - 1 - d // 2, mid + d // 2
        korder.append(hi if d % 2 == 0 else lo)

    for k in korder:
        qzl = qzlo_ref[b, k]
        qzh = qzhi_ref[b, k]
        qs = q[k * _Q_SUB:(k + 1) * _Q_SUB, :]
        jl = jl_ref[b, k]

        acc = macc_ref[...]
        for i in range(_NW):
            acc = block_fold(qs, jl + i, acc)
        macc_ref[...] = acc

        m_ref[0] = jnp.minimum(m_ref[0], jnp.min(acc))

        def body(c, _step=1, _qzl=qzl, _qzh=qzh, _qs=qs):
            j, _ = c
            pzl = pzlo_ref[j]
            pzh = pzhi_ref[j]
            gap = jnp.maximum(jnp.maximum(pzl - _qzh, _qzl - pzh), 0.0)
            pred = gap * gap < m_ref[0]

            @pl.when(pred)
            def _():
                block_min(_qs, j)

            return (j + _step, jnp.logical_not(pred))

        def cond(c):
            j, done = c
            return (j >= 0) & (j < nc) & jnp.logical_not(done)

        jax.lax.while_loop(cond, functools.partial(body, _step=1),
                           (jl + _NW, False))
        jax.lax.while_loop(cond, functools.partial(body, _step=-1),
                           (jl - 1, False))

    out_ref[...] = jnp.full(
        out_ref.shape,
        jnp.sqrt(jnp.maximum(jnp.min(macc_ref[...]), 0.0)) - _RADIUS,
        dtype=out_ref.dtype)


def kernel(x, pcd):
    B, P, _ = x.shape
    N = pcd.shape[0]
    n_pad = -N % _W
    nc = (N + n_pad) // _W
    nsub = P // _Q_SUB

    ps = jax.lax.sort([pcd[:, 2], pcd[:, 0], pcd[:, 1]], num_keys=1)
    pcd_t = jnp.stack([ps[1], ps[2], ps[0]])
    pcd_t = jnp.pad(pcd_t, ((0, 0), (0, n_pad)), mode="edge")
    qs = jax.lax.sort([x[:, :, 2], x[:, :, 0], x[:, :, 1]],
                      dimension=1, num_keys=1)
    x_s = jnp.stack([qs[1], qs[2], qs[0]], axis=-1)

    pz = pcd_t[2]
    pzlo = pz[::_W]
    pzhi = pz[_W - 1::_W]
    qz = x_s[:, :, 2]
    qzlo = qz[:, ::_Q_SUB]
    qzhi = qz[:, _Q_SUB - 1::_Q_SUB]

    qc = 0.5 * (qzlo + qzhi)
    j0 = jnp.minimum(jnp.searchsorted(pzhi, qc.reshape(-1)),
                     nc - 1).astype(jnp.int32).reshape(B, nsub)
    jl = jnp.clip(j0 - _NW // 2, 0, nc - _NW)

    p_chunks = pcd_t.reshape(3, nc, _W).transpose(1, 0, 2)

    grid_spec = pltpu.PrefetchScalarGridSpec(
        num_scalar_prefetch=5,
        grid=(B,),
        in_specs=[
            pl.BlockSpec((1, P, 3), lambda b, *_: (b, 0, 0)),
            pl.BlockSpec((nc, 3, _W), lambda b, *_: (0, 0, 0)),
        ],
        out_specs=pl.BlockSpec((1, 1, 128), lambda b, *_: (b, 0, 0)),
        scratch_shapes=[pltpu.SMEM((1,), jnp.float32),
                        pltpu.VMEM((_Q_SUB, 128), jnp.float32)],
    )
    out = pl.pallas_call(
        functools.partial(_csdf_kernel, nc, nsub),
        grid_spec=grid_spec,
        out_shape=jax.ShapeDtypeStruct((B, 1, 128), jnp.float32),
    )(qzlo, qzhi, pzlo, pzhi, jl, x_s, p_chunks)
    return out[:, 0, 0]


# speedup vs baseline: 2.5341x; 1.2841x over previous
import functools

import jax
import jax.numpy as jnp
from jax.experimental import pallas as pl
from jax.experimental.pallas import tpu as pltpu

_RADIUS = 0.02
_W = 512
_NW = 4
_Q_SUB = 64


def _csdf_kernel(nc, nsub, qzlo_ref, qzhi_ref, pzlo_ref, pzhi_ref, jl_ref,
                 x_ref, p_ref, out_ref, m_ref, macc_ref):
    b = pl.program_id(0)
    m_ref[0] = jnp.inf
    macc_ref[...] = jnp.full(macc_ref.shape, jnp.inf, jnp.float32)
    q = x_ref[0]

    def block_fold(qs, j, acc):
        pt = p_ref[j]
        d2 = None
        for c3 in range(3):
            diff = qs[:, c3:c3 + 1] - pt[c3:c3 + 1, :]
            sq = diff * diff
            d2 = sq if d2 is None else d2 + sq
        for w in range(_W // 128):
            acc = jnp.minimum(acc, d2[:, w * 128:(w + 1) * 128])
        return acc

    def block_min(qs, j):
        macc_ref[...] = block_fold(qs, j, macc_ref[...])

    acc = macc_ref[...]
    for k in range(nsub):
        qs = q[k * _Q_SUB:(k + 1) * _Q_SUB, :]
        jl = jl_ref[b, k]
        for i in range(_NW):
            acc = block_fold(qs, jl + i, acc)
    macc_ref[...] = acc
    m_ref[0] = jnp.min(acc)

    for k in range(nsub):
        qzl = qzlo_ref[b, k]
        qzh = qzhi_ref[b, k]
        qs = q[k * _Q_SUB:(k + 1) * _Q_SUB, :]
        jl = jl_ref[b, k]

        def body(c, _step=1, _qzl=qzl, _qzh=qzh, _qs=qs):
            j, _ = c
            pzl = pzlo_ref[j]
            pzh = pzhi_ref[j]
            gap = jnp.maximum(jnp.maximum(pzl - _qzh, _qzl - pzh), 0.0)
            pred = gap * gap < m_ref[0]

            @pl.when(pred)
            def _():
                block_min(_qs, j)

            return (j + _step, jnp.logical_not(pred))

        def cond(c):
            j, done = c
            return (j >= 0) & (j < nc) & jnp.logical_not(done)

        jax.lax.while_loop(cond, functools.partial(body, _step=1),
                           (jl + _NW, False))
        jax.lax.while_loop(cond, functools.partial(body, _step=-1),
                           (jl - 1, False))

    out_ref[...] = jnp.full(
        out_ref.shape,
        jnp.sqrt(jnp.maximum(jnp.min(macc_ref[...]), 0.0)) - _RADIUS,
        dtype=out_ref.dtype)


def kernel(x, pcd):
    B, P, _ = x.shape
    N = pcd.shape[0]
    n_pad = -N % _W
    nc = (N + n_pad) // _W
    nsub = P // _Q_SUB

    ps = jax.lax.sort([pcd[:, 2], pcd[:, 0], pcd[:, 1]], num_keys=1)
    pcd_t = jnp.stack([ps[1], ps[2], ps[0]])
    pcd_t = jnp.pad(pcd_t, ((0, 0), (0, n_pad)), mode="edge")
    qs = jax.lax.sort([x[:, :, 2], x[:, :, 0], x[:, :, 1]],
                      dimension=1, num_keys=1)
    x_s = jnp.stack([qs[1], qs[2], qs[0]], axis=-1)

    pz = pcd_t[2]
    pzlo = pz[::_W]
    pzhi = pz[_W - 1::_W]
    qz = x_s[:, :, 2]
    qzlo = qz[:, ::_Q_SUB]
    qzhi = qz[:, _Q_SUB - 1::_Q_SUB]

    qc = 0.5 * (qzlo + qzhi)
    j0 = jnp.minimum(jnp.searchsorted(pzhi, qc.reshape(-1)),
                     nc - 1).astype(jnp.int32).reshape(B, nsub)
    jl = jnp.clip(j0 - _NW // 2, 0, nc - _NW)

    p_chunks = pcd_t.reshape(3, nc, _W).transpose(1, 0, 2)

    grid_spec = pltpu.PrefetchScalarGridSpec(
        num_scalar_prefetch=5,
        grid=(B,),
        in_specs=[
            pl.BlockSpec((1, P, 3), lambda b, *_: (b, 0, 0)),
            pl.BlockSpec((nc, 3, _W), lambda b, *_: (0, 0, 0)),
        ],
        out_specs=pl.BlockSpec((1, 1, 128), lambda b, *_: (b, 0, 0)),
        scratch_shapes=[pltpu.SMEM((1,), jnp.float32),
                        pltpu.VMEM((_Q_SUB, 128), jnp.float32)],
    )
    out = pl.pallas_call(
        functools.partial(_csdf_kernel, nc, nsub),
        grid_spec=grid_spec,
        out_shape=jax.ShapeDtypeStruct((B, 1, 128), jnp.float32),
    )(qzlo, qzhi, pzlo, pzhi, jl, x_s, p_chunks)
    return out[:, 0, 0]
